# Initial kernel scaffold; baseline (speedup 1.0000x reference)
import jax
import jax.numpy as jnp
from jax.experimental import pallas as pl


def kernel(u_f, edge_index):
    raise NotImplementedError("write your pallas kernel here")


# speedup vs baseline: 9.7461x; 9.7461x over previous
import functools

import jax
import jax.numpy as jnp
from jax import lax
from jax.experimental import pallas as pl
from jax.experimental.pallas import tpu as pltpu
from jax.experimental.pallas import tpu_sc as plsc

N = 10000
E = 320000
D = 128
NC = 2
NS = 16
NW = NC * NS
EPW = E // NW
CH = 125
NCHUNK = EPW // CH
RPT = 640
NP = NS * RPT
HW = 16

_mesh = plsc.VectorSubcoreMesh(
    core_axis_name="c", subcore_axis_name="s", num_cores=NC, num_subcores=NS)


def _degrees_body(src_hbm, dst_hbm, ones_hbm, zeros_hbm,
                  od_hbm, id_hbm,
                  od_sh, id_sh, idx_s, idx_d, ones_v, sem):
    c = lax.axis_index("c")
    s = lax.axis_index("s")
    wid = c * NS + s
    pltpu.sync_copy(zeros_hbm, od_sh.at[pl.ds(s * RPT, RPT)])
    pltpu.sync_copy(zeros_hbm, id_sh.at[pl.ds(s * RPT, RPT)])
    pltpu.sync_copy(ones_hbm, ones_v)
    pltpu.sync_copy(src_hbm.at[wid], idx_s)
    pltpu.sync_copy(dst_hbm.at[wid], idx_d)
    plsc.subcore_barrier()

    def chunk(j, carry):
        pltpu.sync_copy(ones_v, od_sh.at[idx_s.at[j]], add=True)
        pltpu.sync_copy(ones_v, id_sh.at[idx_d.at[j]], add=True)
        return carry

    lax.fori_loop(0, NCHUNK, chunk, 0)
    plsc.subcore_barrier()
    pltpu.sync_copy(od_sh.at[pl.ds(s * RPT, RPT)],
                    od_hbm.at[pl.ds(c * NP + s * RPT, RPT)])
    pltpu.sync_copy(id_sh.at[pl.ds(s * RPT, RPT)],
                    id_hbm.at[pl.ds(c * NP + s * RPT, RPT)])


_degrees = pl.kernel(
    _degrees_body,
    out_type=(jax.ShapeDtypeStruct((NC * NP,), jnp.float32),
              jax.ShapeDtypeStruct((NC * NP,), jnp.float32)),
    mesh=_mesh,
    scratch_types=[
        pltpu.VMEM_SHARED((NP,), jnp.float32),
        pltpu.VMEM_SHARED((NP,), jnp.float32),
        pltpu.VMEM((NCHUNK, CH), jnp.int32),
        pltpu.VMEM((NCHUNK, CH), jnp.int32),
        pltpu.VMEM((CH,), jnp.float32),
        pltpu.SemaphoreType.DMA,
    ],
)


def _scale_body(uf_ref, od_ref, out_ref):
    deg = od_ref[pl.ds(0, N)] + od_ref[pl.ds(NP, N)]
    norm = lax.rsqrt(jnp.maximum(deg, 1.0))
    out_ref[...] = uf_ref[...] * norm[:, None]


_scale = pl.pallas_call(
    _scale_body,
    out_shape=jax.ShapeDtypeStruct((N, D), jnp.float32),
)


def _push_body(nf_hbm, src_hbm, dst_hbm, zeros_hbm,
               part_hbm,
               acc, idx_s, idx_d, rows, sem):
    c = lax.axis_index("c")
    s = lax.axis_index("s")
    wid = c * NS + s
    pltpu.sync_copy(zeros_hbm, acc.at[pl.ds(s * RPT, RPT)])
    pltpu.sync_copy(src_hbm.at[wid], idx_s)
    pltpu.sync_copy(dst_hbm.at[wid], idx_d)
    plsc.subcore_barrier()

    def chunk(j, carry):
        pltpu.async_copy(nf_hbm.at[idx_s.at[j]], rows, sem).wait()
        pltpu.sync_copy(rows, acc.at[idx_d.at[j]], add=True)
        return carry

    lax.fori_loop(0, NCHUNK, chunk, 0)
    plsc.subcore_barrier()
    pltpu.sync_copy(acc.at[pl.ds(s * RPT, RPT)],
                    part_hbm.at[c, pl.ds(s * RPT, RPT)])


_push = pl.kernel(
    _push_body,
    out_type=jax.ShapeDtypeStruct((NC, NP, D), jnp.float32),
    mesh=_mesh,
    scratch_types=[
        pltpu.VMEM_SHARED((NP, D), jnp.float32),
        pltpu.VMEM((NCHUNK, CH), jnp.int32),
        pltpu.VMEM((NCHUNK, CH), jnp.int32),
        pltpu.VMEM((CH, D), jnp.float32),
        pltpu.SemaphoreType.DMA,
    ],
)


def _combine_body(p_ref, id_ref, out_ref):
    deg = id_ref[pl.ds(0, N)] + id_ref[pl.ds(NP, N)]
    norm = lax.rsqrt(jnp.maximum(deg, 1.0))
    out_ref[...] = (p_ref[0, :N, :] + p_ref[1, :N, :]) * norm[:, None]


_combine = pl.pallas_call(
    _combine_body,
    out_shape=jax.ShapeDtypeStruct((N, D), jnp.float32),
)


@jax.jit
def kernel(u_f, edge_index):
    ei = edge_index.astype(jnp.int32)
    src3 = ei[0].reshape(NW, NCHUNK, CH)
    dst3 = ei[1].reshape(NW, NCHUNK, CH)
    ones_h = jnp.ones((CH,), jnp.float32)
    zeros_h = jnp.zeros((RPT,), jnp.float32)
    zeros_d = jnp.zeros((RPT, D), jnp.float32)
    od_p, id_p = _degrees(src3, dst3, ones_h, zeros_h)
    node_f = _scale(u_f, od_p)
    parts = _push(node_f, src3, dst3, zeros_d)
    return _combine(parts, id_p)
